# baseline (speedup 1.0000x reference)
import functools

import jax
import jax.numpy as jnp
from jax import lax
from jax.experimental import pallas as pl
from jax.experimental.pallas import tpu as pltpu
from jax.experimental.pallas import tpu_sc as plsc

N_TOKENS = 32768
D_MODEL = 768
N_EXPERTS = 64
N_ACTIVE = 8
BLOCK_ROWS = 4096

_NUM_SC = 2
_NUM_SUBCORES = 16
_NW = _NUM_SC * _NUM_SUBCORES
_ROWS_PER_W = N_TOKENS // _NW
_LANES = 16


def _probs_block(x_ref, w_ref, probs_ref):
    x = x_ref[...]
    w = w_ref[...]
    lt = jax.lax.dot_general(
        w, x, (((1,), (1,)), ((), ())), preferred_element_type=jnp.float32
    )
    m = jnp.max(lt, axis=0, keepdims=True)
    et = jnp.exp(lt - m)
    s = jnp.sum(et, axis=0, keepdims=True)
    probs_ref[...] = (et / s).T


def _tc_probs(x, W):
    n = x.shape[0]
    return pl.pallas_call(
        _probs_block,
        grid=(n // BLOCK_ROWS,),
        in_specs=[
            pl.BlockSpec((BLOCK_ROWS, D_MODEL), lambda i: (i, 0)),
            pl.BlockSpec((N_EXPERTS, D_MODEL), lambda i: (0, 0)),
        ],
        out_specs=pl.BlockSpec((BLOCK_ROWS, N_EXPERTS), lambda i: (i, 0)),
        out_shape=jax.ShapeDtypeStruct((n, N_EXPERTS), jnp.float32),
    )(x, W)


def _merge_top8(ak, av, bk, bv, lo8, descending):
    mk = jnp.where(lo8, ak, bk)
    mv = jnp.where(lo8, av, bv)
    return plsc.sort_key_val(mk, mv, descending=descending)


_CHUNK = 512
_N_CHUNKS = _ROWS_PER_W // _CHUNK


def _sc_topk_body(n_tokens, probs_hbm, vals_hbm, idx_hbm, pbuf, vbuf, ibuf):
    rows_per_w = n_tokens // _NW
    n_chunks = rows_per_w // _CHUNK
    wid = lax.axis_index("s") * _NUM_SC + lax.axis_index("c")
    base = wid * rows_per_w

    lane = lax.iota(jnp.int32, _LANES)
    lo8 = lane < N_ACTIVE

    for chunk in range(n_chunks):
        cbase = base + chunk * _CHUNK
        pltpu.sync_copy(probs_hbm.at[pl.ds(cbase, _CHUNK)], pbuf)

        @plsc.parallel_loop(0, _CHUNK)
        def _row(r):
            ks = []
            vs = []
            for c in range(N_EXPERTS // _LANES):
                k = pbuf[r, pl.ds(c * _LANES, _LANES)]
                sk, sv = plsc.sort_key_val(
                    k, lane + c * _LANES, descending=(c % 2 == 0)
                )
                ks.append(sk)
                vs.append(sv)
            k01, v01 = _merge_top8(
                ks[0], vs[0], ks[1], vs[1], lo8, descending=True
            )
            k23, v23 = _merge_top8(
                ks[2], vs[2], ks[3], vs[3], lo8, descending=False
            )
            k8, v8 = _merge_top8(k01, v01, k23, v23, lo8, descending=True)
            s8 = jnp.sum(jnp.where(lo8, k8, 0.0))
            vals = k8 / (s8 + 1e-6)
            off = pl.multiple_of(r * N_ACTIVE, 8)
            plsc.store_compressed(vbuf.at[pl.ds(off, _LANES)], vals, mask=lo8)
            plsc.store_compressed(ibuf.at[pl.ds(off, _LANES)], v8, mask=lo8)

        nout = _CHUNK * N_ACTIVE
        pltpu.sync_copy(
            vbuf.at[pl.ds(0, nout)],
            vals_hbm.at[pl.ds(cbase * N_ACTIVE, nout)],
        )
        pltpu.sync_copy(
            ibuf.at[pl.ds(0, nout)],
            idx_hbm.at[pl.ds(cbase * N_ACTIVE, nout)],
        )


@functools.cache
def _make_sc_topk(n_tokens):
    return functools.partial(
        pl.kernel,
        out_type=[
            jax.ShapeDtypeStruct((n_tokens * N_ACTIVE,), jnp.float32),
            jax.ShapeDtypeStruct((n_tokens * N_ACTIVE,), jnp.int32),
        ],
        mesh=plsc.VectorSubcoreMesh(
            core_axis_name="c", subcore_axis_name="s"
        ),
        compiler_params=pltpu.CompilerParams(needs_layout_passes=False),
        scratch_types=[
            pltpu.VMEM((_CHUNK, N_EXPERTS), jnp.float32),
            pltpu.VMEM((_CHUNK * N_ACTIVE + 8,), jnp.float32),
            pltpu.VMEM((_CHUNK * N_ACTIVE + 8,), jnp.int32),
        ],
    )(functools.partial(_sc_topk_body, n_tokens))


_N_SPLITS = 2


@jax.jit
def kernel(x, W):
    ch = N_TOKENS // _N_SPLITS
    sc_topk = _make_sc_topk(ch)
    probs_parts = []
    vals_parts = []
    idx_parts = []
    for s in range(_N_SPLITS):
        p = _tc_probs(jax.lax.slice_in_dim(x, s * ch, (s + 1) * ch), W)
        vf, if_ = sc_topk(p)
        probs_parts.append(p)
        vals_parts.append(vf.reshape(ch, N_ACTIVE))
        idx_parts.append(if_.reshape(ch, N_ACTIVE))
    probs = jnp.concatenate(probs_parts, axis=0)
    vals = jnp.concatenate(vals_parts, axis=0)
    idx = jnp.concatenate(idx_parts, axis=0)
    return (vals, idx, probs)


# speedup vs baseline: 1.6518x; 1.6518x over previous
import functools

import jax
import jax.numpy as jnp
from jax import lax
from jax.experimental import pallas as pl
from jax.experimental.pallas import tpu as pltpu
from jax.experimental.pallas import tpu_sc as plsc

N_TOKENS = 32768
D_MODEL = 768
N_EXPERTS = 64
N_ACTIVE = 8
BLOCK_ROWS = 4096

_NUM_SC = 2
_NUM_SUBCORES = 16
_NW = _NUM_SC * _NUM_SUBCORES
_ROWS_PER_W = N_TOKENS // _NW
_LANES = 16


def _probs_block(x_ref, w_ref, probs_ref):
    x = x_ref[...]
    w = w_ref[...]
    lt = jax.lax.dot_general(
        w, x, (((1,), (1,)), ((), ())), preferred_element_type=jnp.float32
    )
    m = jnp.max(lt, axis=0, keepdims=True)
    et = jnp.exp(lt - m)
    s = jnp.sum(et, axis=0, keepdims=True)
    probs_ref[...] = (et / s).T


def _tc_probs(x, W):
    n = x.shape[0]
    return pl.pallas_call(
        _probs_block,
        grid=(n // BLOCK_ROWS,),
        in_specs=[
            pl.BlockSpec((BLOCK_ROWS, D_MODEL), lambda i: (i, 0)),
            pl.BlockSpec((N_EXPERTS, D_MODEL), lambda i: (0, 0)),
        ],
        out_specs=pl.BlockSpec((BLOCK_ROWS, N_EXPERTS), lambda i: (i, 0)),
        out_shape=jax.ShapeDtypeStruct((n, N_EXPERTS), jnp.float32),
    )(x, W)


def _merge_top8(ak, av, bk, bv, lo8, descending):
    mk = jnp.where(lo8, ak, bk)
    mv = jnp.where(lo8, av, bv)
    return plsc.sort_key_val(mk, mv, descending=descending)


_CHUNK = 512
_N_CHUNKS = _ROWS_PER_W // _CHUNK


def _sc_topk_body(probs_hbm, vals_hbm, idx_hbm, pbuf, vbuf, ibuf):
    wid = lax.axis_index("s") * _NUM_SC + lax.axis_index("c")
    base = wid * _ROWS_PER_W

    lane = lax.iota(jnp.int32, _LANES)
    lo8 = lane < N_ACTIVE

    for chunk in range(_N_CHUNKS):
        cbase = base + chunk * _CHUNK
        pltpu.sync_copy(probs_hbm.at[pl.ds(cbase, _CHUNK)], pbuf)

        @plsc.parallel_loop(0, _CHUNK)
        def _row(r):
            ks = []
            vs = []
            for c in range(N_EXPERTS // _LANES):
                k = pbuf[r, pl.ds(c * _LANES, _LANES)]
                sk, sv = plsc.sort_key_val(
                    k, lane + c * _LANES, descending=(c % 2 == 0)
                )
                ks.append(sk)
                vs.append(sv)
            k01, v01 = _merge_top8(
                ks[0], vs[0], ks[1], vs[1], lo8, descending=True
            )
            k23, v23 = _merge_top8(
                ks[2], vs[2], ks[3], vs[3], lo8, descending=False
            )
            k8, v8 = _merge_top8(k01, v01, k23, v23, lo8, descending=True)
            s8 = jnp.sum(jnp.where(lo8, k8, 0.0))
            vals = k8 / (s8 + 1e-6)
            off = pl.multiple_of(r * N_ACTIVE, 8)
            plsc.store_compressed(vbuf.at[pl.ds(off, _LANES)], vals, mask=lo8)
            plsc.store_compressed(ibuf.at[pl.ds(off, _LANES)], v8, mask=lo8)

        nout = _CHUNK * N_ACTIVE
        pltpu.sync_copy(
            vbuf.at[pl.ds(0, nout)],
            vals_hbm.at[pl.ds(cbase * N_ACTIVE, nout)],
        )
        pltpu.sync_copy(
            ibuf.at[pl.ds(0, nout)],
            idx_hbm.at[pl.ds(cbase * N_ACTIVE, nout)],
        )


_sc_topk = functools.partial(
    pl.kernel,
    out_type=[
        jax.ShapeDtypeStruct((N_TOKENS * N_ACTIVE,), jnp.float32),
        jax.ShapeDtypeStruct((N_TOKENS * N_ACTIVE,), jnp.int32),
    ],
    mesh=plsc.VectorSubcoreMesh(core_axis_name="c", subcore_axis_name="s"),
    compiler_params=pltpu.CompilerParams(needs_layout_passes=False),
    scratch_types=[
        pltpu.VMEM((_CHUNK, N_EXPERTS), jnp.float32),
        pltpu.VMEM((_CHUNK * N_ACTIVE + 8,), jnp.float32),
        pltpu.VMEM((_CHUNK * N_ACTIVE + 8,), jnp.int32),
    ],
)(_sc_topk_body)


@jax.jit
def kernel(x, W):
    probs = _tc_probs(x, W)
    vals_flat, idx_flat = _sc_topk(probs)
    vals = vals_flat.reshape(N_TOKENS, N_ACTIVE)
    idx = idx_flat.reshape(N_TOKENS, N_ACTIVE)
    return (vals, idx, probs)
